# baseline (speedup 1.0000x reference)
import functools

import numpy as np
import jax
import jax.numpy as jnp
from jax import lax
from jax.experimental import pallas as pl
from jax.experimental.pallas import tpu as pltpu
from jax.experimental.pallas import tpu_sc as plsc

_B, _L, _D = 64, 2048, 64
_N = _B * _L

np.random.seed(0)
_MSHUF = np.random.permutation(np.arange(_N)).astype(np.int32)

_info = plsc.get_sparse_core_info()
_NC, _NS = _info.num_cores, _info.num_subcores
_NW = _NC * _NS
_RPW = _N // _NW
_LB = 256
_NT = _RPW // _LB
_TPB = _L // _LB

_mesh = plsc.VectorSubcoreMesh(core_axis_name="c", subcore_axis_name="s")
_PARAMS = pltpu.CompilerParams(
    use_tc_tiling_on_sc=False, needs_layout_passes=False, skip_device_barrier=True
)

def _lane_consts():
    iota = jnp.arange(16, dtype=jnp.int32)
    col = [iota + 16 * j for j in range(_LB // 16)]
    ltb = [jnp.full((16,), (16 * j) // 128, jnp.int32) for j in range(_LB // 16)]
    li = [iota + (16 * j) % 128 for j in range(_LB // 16)]
    return iota, col, ltb, li


@functools.partial(
    pl.kernel,
    mesh=_mesh,
    compiler_params=_PARAMS,
    out_type=jax.ShapeDtypeStruct((_N, _D), jnp.float32),
    scratch_types=[
        pltpu.VMEM((_N // _NW,), jnp.int32),
        pltpu.VMEM((8, 2, 8, 128), jnp.float32),
        pltpu.VMEM((8, 2, 8, 128), jnp.float32),
        pltpu.VMEM((_LB, _D), jnp.float32),
        pltpu.VMEM((_LB, _D), jnp.float32),
        pltpu.SemaphoreType.DMA,
        pltpu.SemaphoreType.DMA,
        pltpu.SemaphoreType.DMA,
        pltpu.SemaphoreType.DMA,
    ],
)
def _scatter_pass(x4_hbm, scat_hbm, out_hbm, sidx_v, blk0, blk1, rows0, rows1,
                  rsem0, rsem1, ssem0, ssem1):
    wid = lax.axis_index("s") * _NC + lax.axis_index("c")
    blks = (blk0, blk1)
    rows = (rows0, rows1)
    rsems = (rsem0, rsem1)
    ssems = (ssem0, ssem1)
    _iota, col_ids, _ltb, _li = _lane_consts()

    def read_block(t, s):
        b = 2 * wid + (t // _TPB)
        lt0 = 2 * (t % _TPB)
        return pltpu.async_copy(
            x4_hbm.at[pl.ds(b * 8, 8), pl.ds(lt0, 2), :, :], blks[s], rsems[s]
        )

    def transpose_block(s):
        blk, row = blks[s], rows[s]

        @plsc.parallel_loop(0, _D, 1, unroll=2)
        def body(c):
            d_vec = (jnp.full((16,), c, jnp.int32) + _iota) & 63
            dtv = jnp.right_shift(d_vec, 3)
            div = jnp.bitwise_and(d_vec, 7)
            for j in range(_LB // 16):
                v = plsc.load_gather(blk, [dtv, _ltb[j], div, _li[j]])
                plsc.store_scatter(row, [col_ids[j], d_vec], v)

    def scatter_block(t, s):
        return pltpu.async_copy(
            rows[s], out_hbm.at[sidx_v.at[pl.ds(t * _LB, _LB)]], ssems[s]
        )

    read_block(0, 0)
    read_block(1, 1)
    pltpu.sync_copy(scat_hbm.at[pl.ds(wid * _RPW, _RPW)], sidx_v)

    def pair_body(p, carry):
        for b in range(2):
            t = 2 * p + b
            pltpu.make_async_copy(
                x4_hbm.at[pl.ds(0, 8), pl.ds(0, 2), :, :], blks[b], rsems[b]
            ).wait()

            @pl.when(p > 0)
            def _():
                pltpu.make_async_copy(rows[b], out_hbm.at[pl.ds(0, _LB)],
                                      ssems[b]).wait()

            transpose_block(b)
            scatter_block(t, b)

            @pl.when(p < (_NT // 2 - 1))
            def _():
                read_block(t + 2, b)

        return carry

    lax.fori_loop(0, _NT // 2, pair_body, 0)
    pltpu.make_async_copy(rows[0], out_hbm.at[pl.ds(0, _LB)], ssems[0]).wait()
    pltpu.make_async_copy(rows[1], out_hbm.at[pl.ds(0, _LB)], ssems[1]).wait()


@functools.partial(
    pl.kernel,
    mesh=_mesh,
    compiler_params=_PARAMS,
    out_type=jax.ShapeDtypeStruct((512, 16, 8, 128), jnp.float32),
    scratch_types=[
        pltpu.VMEM((_LB, _D), jnp.float32),
        pltpu.VMEM((_LB, _D), jnp.float32),
        pltpu.VMEM((8, 2, 8, 128), jnp.float32),
        pltpu.VMEM((8, 2, 8, 128), jnp.float32),
        pltpu.SemaphoreType.DMA,
        pltpu.SemaphoreType.DMA,
        pltpu.SemaphoreType.DMA,
        pltpu.SemaphoreType.DMA,
    ],
)
def _untranspose_pass(src_hbm, out_hbm, rows0, rows1, blk0, blk1,
                      rsem0, rsem1, wsem0, wsem1):
    wid = lax.axis_index("s") * _NC + lax.axis_index("c")
    rows = (rows0, rows1)
    blks = (blk0, blk1)
    rsems = (rsem0, rsem1)
    wsems = (wsem0, wsem1)
    base = wid * _RPW
    _iota, col_ids, _ltb, _li = _lane_consts()

    def read_block(t, s):
        return pltpu.async_copy(
            src_hbm.at[pl.ds(base + t * _LB, _LB)], rows[s], rsems[s]
        )

    def transpose_block(s):
        row, blk = rows[s], blks[s]

        @plsc.parallel_loop(0, _D, 1, unroll=2)
        def body(c):
            d_vec = (jnp.full((16,), c, jnp.int32) + _iota) & 63
            dtv = jnp.right_shift(d_vec, 3)
            div = jnp.bitwise_and(d_vec, 7)
            for j in range(_LB // 16):
                v = plsc.load_gather(row, [col_ids[j], d_vec])
                plsc.store_scatter(blk, [dtv, _ltb[j], div, _li[j]], v)

    def write_block(t, s):
        b = 2 * wid + (t // _TPB)
        lt0 = 2 * (t % _TPB)
        return pltpu.async_copy(
            blks[s], out_hbm.at[pl.ds(b * 8, 8), pl.ds(lt0, 2), :, :], wsems[s]
        )

    read_block(0, 0)
    read_block(1, 1)

    def pair_body(p, carry):
        for b in range(2):
            t = 2 * p + b
            pltpu.make_async_copy(
                src_hbm.at[pl.ds(0, _LB)], rows[b], rsems[b]
            ).wait()

            @pl.when(p > 0)
            def _():
                pltpu.make_async_copy(
                    blks[b], out_hbm.at[pl.ds(0, 8), pl.ds(0, 2), :, :],
                    wsems[b]
                ).wait()

            transpose_block(b)
            write_block(t, b)

            @pl.when(p < (_NT // 2 - 1))
            def _():
                read_block(t + 2, b)

        return carry

    lax.fori_loop(0, _NT // 2, pair_body, 0)
    pltpu.make_async_copy(blks[0], out_hbm.at[pl.ds(0, 8), pl.ds(0, 2), :, :],
                          wsems[0]).wait()
    pltpu.make_async_copy(blks[1], out_hbm.at[pl.ds(0, 8), pl.ds(0, 2), :, :],
                          wsems[1]).wait()


def kernel(x):
    x4 = (
        x.transpose(0, 2, 1)
        .reshape(_B, 8, 8, _L // 128, 128)
        .transpose(0, 1, 3, 2, 4)
        .reshape(512, _L // 128, 8, 128)
    )
    scat = jnp.asarray(_MSHUF)
    scratch = _scatter_pass(x4, scat)
    y4 = _untranspose_pass(scratch)
    return (
        y4.reshape(_B, 8, _L // 128, 8, 128)
        .transpose(0, 1, 3, 2, 4)
        .reshape(_B, _D, _L)
        .transpose(0, 2, 1)
    )


# speedup vs baseline: 1.0418x; 1.0418x over previous
import functools

import numpy as np
import jax
import jax.numpy as jnp
from jax import lax
from jax.experimental import pallas as pl
from jax.experimental.pallas import tpu as pltpu
from jax.experimental.pallas import tpu_sc as plsc

_B, _L, _D = 64, 2048, 64
_N = _B * _L

np.random.seed(0)
_MSHUF = np.random.permutation(np.arange(_N)).astype(np.int32)

_info = plsc.get_sparse_core_info()
_NC, _NS = _info.num_cores, _info.num_subcores
_NW = _NC * _NS
_RPW = _N // _NW
_LB = 256
_NT = _RPW // _LB
_TPB = _L // _LB
_MAGIC = 0x5CBA17

_mesh = plsc.VectorSubcoreMesh(core_axis_name="c", subcore_axis_name="s")
_PARAMS = pltpu.CompilerParams(
    use_tc_tiling_on_sc=False, needs_layout_passes=False, skip_device_barrier=True
)


def _lane_consts():
    iota = jnp.arange(16, dtype=jnp.int32)
    col = [iota + 16 * j for j in range(_LB // 16)]
    ltb = [jnp.full((16,), (16 * j) // 128, jnp.int32) for j in range(_LB // 16)]
    li = [iota + (16 * j) % 128 for j in range(_LB // 16)]
    return iota, col, ltb, li


@functools.partial(
    pl.kernel,
    mesh=_mesh,
    compiler_params=_PARAMS,
    out_type=(
        jax.ShapeDtypeStruct((512, 16, 8, 128), jnp.float32),
        jax.ShapeDtypeStruct((_N, _D), jnp.float32),
        jax.ShapeDtypeStruct((2, 16), jnp.int32),
    ),
    scratch_types=[
        pltpu.VMEM((_RPW,), jnp.int32),
        pltpu.VMEM((8, 2, 8, 128), jnp.float32),
        pltpu.VMEM((8, 2, 8, 128), jnp.float32),
        pltpu.VMEM((_LB, _D), jnp.float32),
        pltpu.VMEM((_LB, _D), jnp.float32),
        pltpu.VMEM((16,), jnp.int32),
        pltpu.SemaphoreType.DMA,
        pltpu.SemaphoreType.DMA,
        pltpu.SemaphoreType.DMA,
        pltpu.SemaphoreType.DMA,
        pltpu.SemaphoreType.DMA,
    ],
)
def _deinterleave(x4_hbm, scat_hbm, y4_hbm, mid_hbm, flag_hbm,
                  sidx_v, blk0, blk1, rows0, rows1, fbuf,
                  rsem0, rsem1, ssem0, ssem1, fsem):
    cid = lax.axis_index("c")
    sid = lax.axis_index("s")
    wid = sid * _NC + cid
    blks = (blk0, blk1)
    rows = (rows0, rows1)
    rsems = (rsem0, rsem1)
    ssems = (ssem0, ssem1)
    _iota, col_ids, _ltb, _li = _lane_consts()


    def read_block1(t, s):
        b = 2 * wid + (t // _TPB)
        lt0 = 2 * (t % _TPB)
        return pltpu.async_copy(
            x4_hbm.at[pl.ds(b * 8, 8), pl.ds(lt0, 2), :, :], blks[s], rsems[s]
        )

    def transpose1(s):
        blk, row = blks[s], rows[s]

        @plsc.parallel_loop(0, _D, 1, unroll=2)
        def body(c):
            d_vec = (jnp.full((16,), c, jnp.int32) + _iota) & 63
            dtv = jnp.right_shift(d_vec, 3)
            div = jnp.bitwise_and(d_vec, 7)
            for j in range(_LB // 16):
                v = plsc.load_gather(blk, [dtv, _ltb[j], div, _li[j]])
                plsc.store_scatter(row, [col_ids[j], d_vec], v)

    read_block1(0, 0)
    read_block1(1, 1)
    pltpu.sync_copy(scat_hbm.at[pl.ds(wid * _RPW, _RPW)], sidx_v)

    def pair_body1(p, carry):
        for s in range(2):
            t = 2 * p + s
            pltpu.make_async_copy(
                x4_hbm.at[pl.ds(0, 8), pl.ds(0, 2), :, :], blks[s], rsems[s]
            ).wait()

            @pl.when(p > 0)
            def _():
                pltpu.make_async_copy(rows[s], mid_hbm.at[pl.ds(0, _LB)],
                                      ssems[s]).wait()

            transpose1(s)
            pltpu.async_copy(
                rows[s], mid_hbm.at[sidx_v.at[pl.ds(t * _LB, _LB)]], ssems[s]
            )

            @pl.when(p < (_NT // 2 - 1))
            def _():
                read_block1(t + 2, s)

        return carry

    lax.fori_loop(0, _NT // 2, pair_body1, 0)
    pltpu.make_async_copy(rows[0], mid_hbm.at[pl.ds(0, _LB)], ssems[0]).wait()
    pltpu.make_async_copy(rows[1], mid_hbm.at[pl.ds(0, _LB)], ssems[1]).wait()


    plsc.subcore_barrier()

    @pl.when(sid == 0)
    def _():
        fbuf[...] = jnp.full((16,), _MAGIC, jnp.int32)
        pltpu.async_copy(fbuf, flag_hbm.at[cid], fsem).wait()

    def poll_cond(c):
        return c != _MAGIC

    def poll_body(c):
        pltpu.sync_copy(flag_hbm.at[1 - cid], fbuf)
        return jnp.max(fbuf[...])

    lax.while_loop(poll_cond, poll_body, jnp.int32(0))
    plsc.subcore_barrier()

    @pl.when(sid == 0)
    def _():
        fbuf[...] = jnp.zeros((16,), jnp.int32)
        pltpu.async_copy(fbuf, flag_hbm.at[1 - cid], fsem).wait()


    base = wid * _RPW

    def read_block2(t, s):
        return pltpu.async_copy(
            mid_hbm.at[pl.ds(base + t * _LB, _LB)], rows[s], rsems[s]
        )

    def transpose2(s):
        row, blk = rows[s], blks[s]

        @plsc.parallel_loop(0, _D, 1, unroll=2)
        def body(c):
            d_vec = (jnp.full((16,), c, jnp.int32) + _iota) & 63
            dtv = jnp.right_shift(d_vec, 3)
            div = jnp.bitwise_and(d_vec, 7)
            for j in range(_LB // 16):
                v = plsc.load_gather(row, [col_ids[j], d_vec])
                plsc.store_scatter(blk, [dtv, _ltb[j], div, _li[j]], v)

    read_block2(0, 0)
    read_block2(1, 1)

    def pair_body2(p, carry):
        for s in range(2):
            t = 2 * p + s
            pltpu.make_async_copy(
                mid_hbm.at[pl.ds(0, _LB)], rows[s], rsems[s]
            ).wait()

            @pl.when(p > 0)
            def _():
                pltpu.make_async_copy(
                    blks[s], y4_hbm.at[pl.ds(0, 8), pl.ds(0, 2), :, :],
                    ssems[s]
                ).wait()

            transpose2(s)
            b = 2 * wid + (t // _TPB)
            lt0 = 2 * (t % _TPB)
            pltpu.async_copy(
                blks[s], y4_hbm.at[pl.ds(b * 8, 8), pl.ds(lt0, 2), :, :],
                ssems[s]
            )

            @pl.when(p < (_NT // 2 - 1))
            def _():
                read_block2(t + 2, s)

        return carry

    lax.fori_loop(0, _NT // 2, pair_body2, 0)
    pltpu.make_async_copy(blks[0], y4_hbm.at[pl.ds(0, 8), pl.ds(0, 2), :, :],
                          ssems[0]).wait()
    pltpu.make_async_copy(blks[1], y4_hbm.at[pl.ds(0, 8), pl.ds(0, 2), :, :],
                          ssems[1]).wait()


def kernel(x):
    x4 = (
        x.transpose(0, 2, 1)
        .reshape(_B, 8, 8, _L // 128, 128)
        .transpose(0, 1, 3, 2, 4)
        .reshape(512, _L // 128, 8, 128)
    )
    scat = jnp.asarray(_MSHUF)
    y4, _mid, _flags = _deinterleave(x4, scat)
    return (
        y4.reshape(_B, 8, _L // 128, 8, 128)
        .transpose(0, 1, 3, 2, 4)
        .reshape(_B, _D, _L)
        .transpose(0, 2, 1)
    )
